# Initial kernel scaffold; baseline (speedup 1.0000x reference)
import jax
import jax.numpy as jnp
from jax.experimental import pallas as pl


def kernel(x, codebook):
    raise NotImplementedError("write your pallas kernel here")


# speedup vs baseline: 2.5583x; 2.5583x over previous
import functools

import jax
import jax.numpy as jnp
from jax.experimental import pallas as pl
from jax.experimental.pallas import tpu as pltpu

N = 262144
D = 64
K = 512
COMMITMENT_COST = 0.25
EPS = 1e-12

B = 2048
NB = N // B


def _vq_body(x_ref, cb_ref, cbt_ref, q_ref, idx_ref, loss_ref):
    step = pl.program_id(0)

    cb = cb_ref[...]
    cbt = cbt_ref[...]
    cb_n = jnp.sqrt(jnp.sum(cb * cb, axis=1, keepdims=True))
    cbn = cb / jnp.maximum(cb_n, EPS)
    cbt_n = jnp.sqrt(jnp.sum(cbt * cbt, axis=0, keepdims=True))
    cbnt = cbt / jnp.maximum(cbt_n, EPS)
    s = jnp.sum(cbnt * cbnt, axis=0, keepdims=True)

    x = x_ref[...]
    xn_norm = jnp.sqrt(jnp.sum(x * x, axis=1, keepdims=True))
    xn = x / jnp.maximum(xn_norm, EPS)
    xsq = jnp.sum(xn * xn, axis=1, keepdims=True)

    dots = jax.lax.dot_general(
        xn, cbnt, (((1,), (0,)), ((), ())),
        preferred_element_type=jnp.float32)
    dist = xsq + s - 2.0 * dots
    dmin = jnp.min(dist, axis=1, keepdims=True)
    col = jax.lax.broadcasted_iota(jnp.int32, (B, K), 1)
    idx = jnp.min(jnp.where(dist == dmin, col, K), axis=1)
    idx_ref[0, 0, :] = idx

    oh = (col == idx[:, None]).astype(jnp.float32)
    q = jax.lax.dot_general(
        oh, cbn, (((1,), (0,)), ((), ())),
        preferred_element_type=jnp.float32)
    q_ref[...] = q

    part = jnp.sum((q - xn) ** 2)

    @pl.when(step == 0)
    def _():
        loss_ref[0, 0] = 0.0

    loss_ref[0, 0] += part


@jax.jit
def kernel(x, codebook):
    q, idx3, loss_sum = pl.pallas_call(
        _vq_body,
        grid=(NB,),
        in_specs=[
            pl.BlockSpec((B, D), lambda i: (i, 0)),
            pl.BlockSpec((K, D), lambda i: (0, 0)),
            pl.BlockSpec((D, K), lambda i: (0, 0)),
        ],
        out_specs=[
            pl.BlockSpec((B, D), lambda i: (i, 0)),
            pl.BlockSpec((1, 1, B), lambda i: (i, 0, 0)),
            pl.BlockSpec((1, 1), lambda i: (0, 0), memory_space=pltpu.SMEM),
        ],
        out_shape=[
            jax.ShapeDtypeStruct((N, D), jnp.float32),
            jax.ShapeDtypeStruct((NB, 1, B), jnp.int32),
            jax.ShapeDtypeStruct((1, 1), jnp.float32),
        ],
    )(x, codebook, codebook.T)
    loss = (loss_sum * ((1.0 + COMMITMENT_COST) / (N * D))).reshape(())
    return q, loss, idx3.reshape(N)
